# baseline (speedup 1.0000x reference)
import functools

import jax
import jax.numpy as jnp
from jax import lax
from jax.experimental import pallas as pl
from jax.experimental.pallas import tpu as pltpu
from jax.experimental.pallas import tpu_sc as plsc

BATCH = 16384
EMBED = 32
HIDDEN = 256
NUM_USERS = 1000000
NUM_MOVIES = 100000
_PACK = 128 // EMBED

_NC, _NS = 2, 16
_NW = _NC * _NS
_B_PER_W = BATCH // _NW
_OROWS = _B_PER_W * EMBED // 128


def _sc_gather(user_id, movie_id, utflat, mtflat):
    mesh = plsc.VectorSubcoreMesh(core_axis_name="c", subcore_axis_name="s")

    @functools.partial(
        pl.kernel,
        mesh=mesh,
        out_type=[
            pltpu.HBM((_NW, _OROWS, 128), jnp.float32),
            pltpu.HBM((_NW, _OROWS, 128), jnp.float32),
        ],
        scratch_types=[
            pltpu.VMEM((_B_PER_W,), jnp.int32),
            pltpu.VMEM((_B_PER_W,), jnp.int32),
            pltpu.VMEM((_OROWS, 128), jnp.int32),
            pltpu.VMEM((_OROWS, 128), jnp.int32),
            pltpu.VMEM((_OROWS, 128), jnp.float32),
            pltpu.VMEM((_OROWS, 128), jnp.float32),
            pltpu.SemaphoreType.DMA,
            pltpu.SemaphoreType.DMA,
        ],
        compiler_params=pltpu.CompilerParams(needs_layout_passes=False),
    )
    def k(uid_hbm, mid_hbm, utab_hbm, mtab_hbm, uout_hbm, mout_hbm,
          uidx_v, midx_v, uoff_v, moff_v,
          uout_v, mout_v, usem, msem):
        wid = lax.axis_index("s") * _NC + lax.axis_index("c")
        base = wid * _B_PER_W
        pltpu.sync_copy(uid_hbm.at[pl.ds(base, _B_PER_W)], uidx_v)
        pltpu.sync_copy(mid_hbm.at[pl.ds(base, _B_PER_W)], midx_v)

        lane = lax.iota(jnp.int32, 16)

        def off_group(k16, _):
            i16 = lane + k16 * 16
            base16 = i16 * EMBED
            u16 = uidx_v[pl.ds(k16 * 16, 16)]
            m16 = midx_v[pl.ds(k16 * 16, 16)]
            for c in range(EMBED):
                pos = base16 + c
                row = lax.shift_right_logical(pos, 7)
                ln = lax.bitwise_and(pos, 127)
                plsc.store_scatter(uoff_v, [row, ln], u16 + c * NUM_USERS)
                plsc.store_scatter(moff_v, [row, ln], m16 + c * NUM_MOVIES)
            return _

        lax.fori_loop(0, _B_PER_W // 16, off_group, 0)

        def gather_body(j, _):
            pltpu.async_copy(utab_hbm.at[uoff_v.at[j]], uout_v.at[j], usem)
            pltpu.async_copy(mtab_hbm.at[moff_v.at[j]], mout_v.at[j], msem)
            return _

        lax.fori_loop(0, _OROWS, gather_body, 0)
        pltpu.make_async_copy(uout_hbm.at[wid], uout_v, usem).wait()
        pltpu.make_async_copy(mout_hbm.at[wid], mout_v, msem).wait()
        pltpu.sync_copy(uout_v, uout_hbm.at[wid])
        pltpu.sync_copy(mout_v, mout_hbm.at[wid])

    return k(user_id, movie_id, utflat, mtflat)


def _mlp_body(u4_ref, m4_ref, w1u_ref, w1m_ref, b1_ref, w2_ref, b2_ref,
              o_ref):
    u4 = u4_ref[...]
    m4 = m4_ref[...]
    outs = []
    for r in range(_PACK):
        sl = slice(r * EMBED, (r + 1) * EMBED)
        x = (jnp.dot(u4[:, sl], w1u_ref[...],
                     preferred_element_type=jnp.float32)
             + jnp.dot(m4[:, sl], w1m_ref[...],
                       preferred_element_type=jnp.float32)
             + b1_ref[...])
        h = jnp.maximum(x, 0.0)
        outs.append(jnp.dot(h, w2_ref[...],
                            preferred_element_type=jnp.float32))
    o_ref[...] = jnp.concatenate(outs, axis=1) + b2_ref[...]


def _tc_mlp(u4, m4, W1u, W1m, b1, W2, b2, block_m=512):
    grid = (BATCH // _PACK // block_m,)
    return pl.pallas_call(
        _mlp_body,
        grid=grid,
        in_specs=[
            pl.BlockSpec((block_m, 128), lambda i: (i, 0)),
            pl.BlockSpec((block_m, 128), lambda i: (i, 0)),
            pl.BlockSpec((EMBED, HIDDEN), lambda i: (0, 0)),
            pl.BlockSpec((EMBED, HIDDEN), lambda i: (0, 0)),
            pl.BlockSpec((1, HIDDEN), lambda i: (0, 0)),
            pl.BlockSpec((HIDDEN, 1), lambda i: (0, 0)),
            pl.BlockSpec((1, _PACK), lambda i: (0, 0)),
        ],
        out_specs=pl.BlockSpec((block_m, _PACK), lambda i: (i, 0)),
        out_shape=jax.ShapeDtypeStruct((BATCH // _PACK, _PACK), jnp.float32),
    )(u4, m4, W1u, W1m, b1, W2, b2)


def kernel(user_id, movie_title, user_table, movie_table, W1, b1, W2, b2):
    uid = user_id.astype(jnp.int32)
    mid = movie_title.astype(jnp.int32)
    utflat = user_table.T.reshape(-1)
    mtflat = movie_table.T.reshape(-1)
    uout, mout = _sc_gather(uid, mid, utflat, mtflat)
    u4 = uout.reshape(BATCH // _PACK, 128)
    m4 = mout.reshape(BATCH // _PACK, 128)
    W1u = W1[:EMBED]
    W1m = W1[EMBED:]
    b2x = jnp.broadcast_to(b2.reshape(1, 1), (1, _PACK))
    o4 = _tc_mlp(u4, m4, W1u, W1m, b1.reshape(1, HIDDEN), W2, b2x)
    return o4.reshape(BATCH, 1)
